# baseline (speedup 1.0000x reference)
import functools

import jax
import jax.numpy as jnp
from jax import lax
from jax.experimental import pallas as pl
from jax.experimental.pallas import tpu as pltpu
from jax.experimental.pallas import tpu_sc as plsc

_NC, _NS, _L = 2, 16, 16
_NW = _NC * _NS

_T, _K, _D = 16384, 9, 32
_R = _K - 1
_V = 1000000
_TW = _T // _NW
_TI = _TW // _L

_SB = 2048
_GB = (_V + _SB - 1) // _SB


def _retile_body(x_ref, o_ref):
    x = x_ref[...]
    o_ref[...] = jnp.transpose(
        x.reshape(_D, _SB // 4, 4), (1, 2, 0)
    ).reshape(_SB // 4, 4 * _D)


def _retile(zt):
    return pl.pallas_call(
        _retile_body,
        grid=(_GB,),
        in_specs=[pl.BlockSpec((_D, _SB), lambda i: (0, i))],
        out_specs=pl.BlockSpec((_SB // 4, 4 * _D), lambda i: (i, 0)),
        out_shape=jax.ShapeDtypeStruct((_V // 4, 4 * _D), jnp.float32),
    )(zt)


_GT = 32 * _TW


def _sc_body(ss_hbm, zf_hbm, outq_hbm, outr_hbm, block_v, idx_v, data_v,
             sem):
    wid = lax.axis_index("s") * _NC + lax.axis_index("c")
    t0 = wid * _TW
    pltpu.sync_copy(ss_hbm.at[pl.ds(t0 * _K, _TW * _K)], block_v)

    lane = jnp.arange(16, dtype=jnp.int32)

    def build_q(ti, carry):
        src = (ti * _L + lane) * _K
        idv = plsc.load_gather(block_v, [src])
        base = idv * _D
        for d in range(_D):
            idx_v[pl.ds(d * _TW + ti * _L, _L)] = base + d
        return carry

    def make_build_r(d0):
        def build_r(ti, carry):
            tk = (ti * _L + lane) * _K
            for r in range(_R):
                idv = plsc.load_gather(block_v, [tk + r + 1])
                base = idv * _D
                for dd in range(4):
                    j = dd * _R + r
                    idx_v[pl.ds(j * _TW + ti * _L, _L)] = base + d0 + dd
            return carry

        return build_r

    def fire(k, carry):
        jrow = k >> 2
        kk = k & 3
        pltpu.async_copy(
            zf_hbm.at[idx_v.at[pl.ds(k * 128, 128)]],
            data_v.at[jrow, pl.ds(kk * 128, 128)],
            sem,
        )
        return carry

    for g in range(_K):
        if g == 0:
            lax.fori_loop(0, _TI, build_q, 0)
        else:
            lax.fori_loop(0, _TI, make_build_r((g - 1) * 4), 0)
        lax.fori_loop(0, _GT // 128, fire, 0)
        pltpu.make_async_copy(
            outr_hbm.at[pl.ds(0, 32), pl.ds(0, _TW)], data_v, sem
        ).wait()
        if g == 0:
            pltpu.sync_copy(data_v, outq_hbm.at[:, pl.ds(t0, _TW)])
        else:
            pltpu.sync_copy(
                data_v,
                outr_hbm.at[pl.ds((g - 1) * 32, 32), pl.ds(t0, _TW)],
            )


@jax.jit
def _run(ss_flat, zt):
    z4 = _retile(zt)
    kfn = pl.kernel(
        _sc_body,
        out_type=(
            jax.ShapeDtypeStruct((_D, _T), jnp.float32),
            jax.ShapeDtypeStruct((_D * _R, _T), jnp.float32),
        ),
        mesh=plsc.VectorSubcoreMesh(
            core_axis_name="c", subcore_axis_name="s",
            num_cores=_NC, num_subcores=_NS,
        ),
        scratch_types=[
            pltpu.VMEM((_TW * _K,), jnp.int32),
            pltpu.VMEM((_GT,), jnp.int32),
            pltpu.VMEM((32, _TW), jnp.float32),
            pltpu.SemaphoreType.DMA,
        ],
        compiler_params=pltpu.CompilerParams(needs_layout_passes=False),
    )
    return kfn(ss_flat, z4.reshape(-1))


def kernel(stimulus_set, max_n_reference, z):
    del max_n_reference
    q2, r2 = _run(stimulus_set.reshape(-1), jnp.transpose(z))
    zq = jnp.transpose(q2).reshape(_T, _D, 1)
    zr = jnp.transpose(r2.reshape(_D, _R, _T), (2, 0, 1))
    return zq, zr


# speedup vs baseline: 1.0452x; 1.0452x over previous
import functools

import jax
import jax.numpy as jnp
from jax import lax
from jax.experimental import pallas as pl
from jax.experimental.pallas import tpu as pltpu
from jax.experimental.pallas import tpu_sc as plsc

_NC, _NS, _L = 2, 16, 16
_NW = _NC * _NS

_T, _K, _D = 16384, 9, 32
_R = _K - 1
_V = 1000000
_TW = _T // _NW
_TI = _TW // _L

_VP = 1000064
_SB = 8192
_GB = (_VP + _SB - 1) // _SB


def _pad_body(x_ref, o_ref):
    o_ref[...] = x_ref[...]


def _pad_copy(zt):
    return pl.pallas_call(
        _pad_body,
        grid=(_GB,),
        in_specs=[pl.BlockSpec((_D, _SB), lambda i: (0, i))],
        out_specs=pl.BlockSpec((_D, _SB), lambda i: (0, i)),
        out_shape=jax.ShapeDtypeStruct((_D, _VP), jnp.float32),
    )(zt)


_GT = 32 * _TW


def _sc_body(ss_hbm, zf_hbm, outq_hbm, outr_hbm, block_v, idx_v, data_v,
             sem):
    wid = lax.axis_index("s") * _NC + lax.axis_index("c")
    t0 = wid * _TW
    pltpu.sync_copy(ss_hbm.at[pl.ds(t0 * _K, _TW * _K)], block_v)

    lane = jnp.arange(16, dtype=jnp.int32)

    def build_q(ti, carry):
        src = (ti * _L + lane) * _K
        idv = plsc.load_gather(block_v, [src])
        for d in range(_D):
            idx_v[pl.ds(d * _TW + ti * _L, _L)] = idv + d * _VP
        return carry

    def make_build_r(d0):
        def build_r(ti, carry):
            tk = (ti * _L + lane) * _K
            for r in range(_R):
                idv = plsc.load_gather(block_v, [tk + r + 1])
                for dd in range(4):
                    j = dd * _R + r
                    idx_v[pl.ds(j * _TW + ti * _L, _L)] = (
                        idv + (d0 + dd) * _VP
                    )
            return carry

        return build_r

    def fire(k, carry):
        jrow = k >> 2
        kk = k & 3
        pltpu.async_copy(
            zf_hbm.at[idx_v.at[pl.ds(k * 128, 128)]],
            data_v.at[jrow, pl.ds(kk * 128, 128)],
            sem,
        )
        return carry

    for g in range(_K):
        if g == 0:
            lax.fori_loop(0, _TI, build_q, 0)
        else:
            lax.fori_loop(0, _TI, make_build_r((g - 1) * 4), 0)
        lax.fori_loop(0, _GT // 128, fire, 0)
        pltpu.make_async_copy(
            outr_hbm.at[pl.ds(0, 32), pl.ds(0, _TW)], data_v, sem
        ).wait()
        if g == 0:
            pltpu.sync_copy(data_v, outq_hbm.at[:, pl.ds(t0, _TW)])
        else:
            pltpu.sync_copy(
                data_v,
                outr_hbm.at[pl.ds((g - 1) * 32, 32), pl.ds(t0, _TW)],
            )


@jax.jit
def _run(ss_flat, zt):
    zp = _pad_copy(zt)
    kfn = pl.kernel(
        _sc_body,
        out_type=(
            jax.ShapeDtypeStruct((_D, _T), jnp.float32),
            jax.ShapeDtypeStruct((_D * _R, _T), jnp.float32),
        ),
        mesh=plsc.VectorSubcoreMesh(
            core_axis_name="c", subcore_axis_name="s",
            num_cores=_NC, num_subcores=_NS,
        ),
        scratch_types=[
            pltpu.VMEM((_TW * _K,), jnp.int32),
            pltpu.VMEM((_GT,), jnp.int32),
            pltpu.VMEM((32, _TW), jnp.float32),
            pltpu.SemaphoreType.DMA,
        ],
        compiler_params=pltpu.CompilerParams(needs_layout_passes=False),
    )
    return kfn(ss_flat, zp.reshape(-1))


def kernel(stimulus_set, max_n_reference, z):
    del max_n_reference
    q2, r2 = _run(stimulus_set.reshape(-1), jnp.transpose(z))
    zq = jnp.transpose(q2).reshape(_T, _D, 1)
    zr = jnp.transpose(r2.reshape(_D, _R, _T), (2, 0, 1))
    return zq, zr


# speedup vs baseline: 4.1254x; 3.9469x over previous
import functools

import jax
import jax.numpy as jnp
from jax import lax
from jax.experimental import pallas as pl
from jax.experimental.pallas import tpu as pltpu
from jax.experimental.pallas import tpu_sc as plsc

_NC, _NS, _L = 2, 16, 16
_NW = _NC * _NS

_T, _K, _D = 16384, 9, 32
_R = _K - 1
_V = 1000000
_TW = _T // _NW
_TI = _TW // _L

_SB = 2048
_GB = (_V + _SB - 1) // _SB


def _retile_body(x_ref, o_ref):
    y = x_ref[...].T
    o_ref[...] = jnp.concatenate(
        [y[k * 512:(k + 1) * 512] for k in range(4)], axis=1
    )


def _retile(zt):
    return pl.pallas_call(
        _retile_body,
        grid=(_GB,),
        in_specs=[pl.BlockSpec((_D, _SB), lambda i: (0, i))],
        out_specs=pl.BlockSpec((_SB // 4, 4 * _D), lambda i: (i, 0)),
        out_shape=jax.ShapeDtypeStruct((_GB * _SB // 4, 4 * _D), jnp.float32),
    )(zt)


_GT = 32 * _TW


def _sc_body(ss_hbm, zf_hbm, outq_hbm, outr_hbm, block_v, idx_v, data_v,
             sem):
    wid = lax.axis_index("s") * _NC + lax.axis_index("c")
    t0 = wid * _TW
    pltpu.sync_copy(ss_hbm.at[pl.ds(t0 * _K, _TW * _K)], block_v)

    lane = jnp.arange(16, dtype=jnp.int32)

    def haddr(s):
        return ((((s >> 11) << 9) + (s & 511)) << 7) + (((s >> 9) & 3) << 5)

    def build_q(ti, carry):
        src = (ti * _L + lane) * _K
        idv = plsc.load_gather(block_v, [src])
        base = haddr(idv)
        for d in range(_D):
            idx_v[pl.ds(d * _TW + ti * _L, _L)] = base + d
        return carry

    def make_build_r(d0):
        def build_r(ti, carry):
            tk = (ti * _L + lane) * _K
            for r in range(_R):
                idv = plsc.load_gather(block_v, [tk + r + 1])
                base = haddr(idv)
                for dd in range(4):
                    j = dd * _R + r
                    idx_v[pl.ds(j * _TW + ti * _L, _L)] = base + d0 + dd
            return carry

        return build_r

    def fire(k, carry):
        jrow = k >> 2
        kk = k & 3
        pltpu.async_copy(
            zf_hbm.at[idx_v.at[pl.ds(k * 128, 128)]],
            data_v.at[jrow, pl.ds(kk * 128, 128)],
            sem,
        )
        return carry

    for g in range(_K):
        if g == 0:
            lax.fori_loop(0, _TI, build_q, 0)
        else:
            lax.fori_loop(0, _TI, make_build_r((g - 1) * 4), 0)
        lax.fori_loop(0, _GT // 128, fire, 0)
        pltpu.make_async_copy(
            outr_hbm.at[pl.ds(0, 32), pl.ds(0, _TW)], data_v, sem
        ).wait()
        if g == 0:
            pltpu.sync_copy(data_v, outq_hbm.at[:, pl.ds(t0, _TW)])
        else:
            pltpu.sync_copy(
                data_v,
                outr_hbm.at[pl.ds((g - 1) * 32, 32), pl.ds(t0, _TW)],
            )


@jax.jit
def _run(ss_flat, zt):
    z4 = _retile(zt)
    kfn = pl.kernel(
        _sc_body,
        out_type=(
            jax.ShapeDtypeStruct((_D, _T), jnp.float32),
            jax.ShapeDtypeStruct((_D * _R, _T), jnp.float32),
        ),
        mesh=plsc.VectorSubcoreMesh(
            core_axis_name="c", subcore_axis_name="s",
            num_cores=_NC, num_subcores=_NS,
        ),
        scratch_types=[
            pltpu.VMEM((_TW * _K,), jnp.int32),
            pltpu.VMEM((_GT,), jnp.int32),
            pltpu.VMEM((32, _TW), jnp.float32),
            pltpu.SemaphoreType.DMA,
        ],
        compiler_params=pltpu.CompilerParams(needs_layout_passes=False),
    )
    return kfn(ss_flat, z4.reshape(-1))


def kernel(stimulus_set, max_n_reference, z):
    del max_n_reference
    q2, r2 = _run(stimulus_set.reshape(-1), jnp.transpose(z))
    zq = jnp.transpose(q2).reshape(_T, _D, 1)
    zr = jnp.transpose(r2.reshape(_D, _R, _T), (2, 0, 1))
    return zq, zr


# speedup vs baseline: 4.3957x; 1.0655x over previous
import functools

import jax
import jax.numpy as jnp
from jax import lax
from jax.experimental import pallas as pl
from jax.experimental.pallas import tpu as pltpu
from jax.experimental.pallas import tpu_sc as plsc

_NC, _NS, _L = 2, 16, 16
_NW = _NC * _NS

_T, _K, _D = 16384, 9, 32
_R = _K - 1
_V = 1000000
_TW = _T // _NW
_TI = _TW // _L

_SB = 2048
_GB = (_V + _SB - 1) // _SB


def _retile_body(x_ref, o_ref):
    y = jax.lax.dot_general(
        x_ref[...], jnp.eye(_D, dtype=jnp.float32),
        (((0,), (0,)), ((), ())),
        preferred_element_type=jnp.float32,
    )
    o_ref[...] = jnp.concatenate(
        [y[k * 512:(k + 1) * 512] for k in range(4)], axis=1
    )


def _retile(zt):
    return pl.pallas_call(
        _retile_body,
        grid=(_GB,),
        in_specs=[pl.BlockSpec((_D, _SB), lambda i: (0, i))],
        out_specs=pl.BlockSpec((_SB // 4, 4 * _D), lambda i: (i, 0)),
        out_shape=jax.ShapeDtypeStruct((_GB * _SB // 4, 4 * _D), jnp.float32),
    )(zt)


_CH = 64
_NCHUNK = _TW // _CH
_ROWS = _CH * _K


def _sc_body(ss_hbm, z4_hbm, outq_hbm, outr_hbm, block_v, srow_v, g_v,
             q_v, o_v, sem):
    wid = lax.axis_index("s") * _NC + lax.axis_index("c")
    t0 = wid * _TW
    pltpu.sync_copy(ss_hbm.at[pl.ds(t0 * _K, _TW * _K)], block_v)

    lane = jnp.arange(16, dtype=jnp.int32)

    def sid_body(i, carry):
        ids = block_v[pl.ds(i * _L, _L)]
        srow_v[pl.ds(i * _L, _L)] = ((ids >> 11) << 9) + (ids & 511)
        return carry

    lax.fori_loop(0, (_TW * _K) // _L, sid_body, 0)

    row_pat = 1 + (lane & 7)
    d_pat = [2 * v + (lane >> 3) for v in range(16)]
    j_pat = [v * _L + lane for v in range(16)]

    for c in range(_NCHUNK // 2):
        for h in range(2):
            base = (c * 2 + h) * _ROWS
            for k in range(4):
                idx = srow_v.at[pl.ds(base + k * 128, 128)]
                pltpu.async_copy(
                    z4_hbm.at[idx], g_v.at[pl.ds(k * 128, 128)], sem
                )
            idx = srow_v.at[pl.ds(base + 512, 64)]
            pltpu.async_copy(z4_hbm.at[idx], g_v.at[pl.ds(512, 64)], sem)
            pltpu.make_async_copy(
                z4_hbm.at[pl.ds(0, _ROWS)], g_v, sem
            ).wait()

            def trial_body(tl, carry):
                g_base = tl * _K
                id_base = base + g_base
                rid = plsc.load_gather(block_v, [id_base + row_pat])
                qid = plsc.load_gather(block_v, [id_base + (lane & 0)])
                rcol = ((rid >> 9) & 3) * _D
                qcol = ((qid >> 9) & 3) * _D
                col = (lane & 0) + tl + h * _CH
                for v in range(_D // _L):
                    vals = plsc.load_gather(
                        g_v, [g_base + (lane & 0), qcol + v * _L + lane]
                    )
                    plsc.store_scatter(q_v, [v * _L + lane, col], vals)
                for v in range(16):
                    vals = plsc.load_gather(
                        g_v, [g_base + row_pat, rcol + d_pat[v]]
                    )
                    plsc.store_scatter(o_v, [j_pat[v], col], vals)
                return carry

            lax.fori_loop(0, _CH, trial_body, 0)

        tc0 = t0 + c * 2 * _CH
        pltpu.sync_copy(q_v, outq_hbm.at[:, pl.ds(tc0, 2 * _CH)])
        pltpu.sync_copy(o_v, outr_hbm.at[:, pl.ds(tc0, 2 * _CH)])


@jax.jit
def _run(ss_flat, zt):
    z4 = _retile(zt)
    kfn = pl.kernel(
        _sc_body,
        out_type=(
            jax.ShapeDtypeStruct((_D, _T), jnp.float32),
            jax.ShapeDtypeStruct((_D * _R, _T), jnp.float32),
        ),
        mesh=plsc.VectorSubcoreMesh(
            core_axis_name="c", subcore_axis_name="s",
            num_cores=_NC, num_subcores=_NS,
        ),
        scratch_types=[
            pltpu.VMEM((_TW * _K,), jnp.int32),
            pltpu.VMEM((_TW * _K,), jnp.int32),
            pltpu.VMEM((_ROWS, 4 * _D), jnp.float32),
            pltpu.VMEM((_D, 2 * _CH), jnp.float32),
            pltpu.VMEM((_D * _R, 2 * _CH), jnp.float32),
            pltpu.SemaphoreType.DMA,
        ],
        compiler_params=pltpu.CompilerParams(needs_layout_passes=False),
    )
    return kfn(ss_flat, z4)


def kernel(stimulus_set, max_n_reference, z):
    del max_n_reference
    q2, r2 = _run(stimulus_set.reshape(-1), jnp.transpose(z))
    zq = jnp.transpose(q2).reshape(_T, _D, 1)
    zr = jnp.transpose(r2.reshape(_D, _R, _T), (2, 0, 1))
    return zq, zr


# speedup vs baseline: 5.9511x; 1.3538x over previous
import functools

import jax
import jax.numpy as jnp
from jax import lax
from jax.experimental import pallas as pl
from jax.experimental.pallas import tpu as pltpu
from jax.experimental.pallas import tpu_sc as plsc

_NC, _NS, _L = 2, 16, 16
_NW = _NC * _NS

_T, _K, _D = 16384, 9, 32
_R = _K - 1
_V = 1000000
_TW = _T // _NW
_TI = _TW // _L

_SB = 8192
_GB = (_V + _SB - 1) // _SB


def _retile_body(x_ref, o_ref):
    y = jax.lax.dot_general(
        x_ref[...], jnp.eye(_D, dtype=jnp.float32),
        (((0,), (0,)), ((), ())),
        preferred_element_type=jnp.float32,
    )
    o_ref[...] = jnp.concatenate(
        [y[k * (_SB // 4):(k + 1) * (_SB // 4)] for k in range(4)], axis=1
    )


def _retile(zt):
    return pl.pallas_call(
        _retile_body,
        grid=(_GB,),
        in_specs=[pl.BlockSpec((_D, _SB), lambda i: (0, i))],
        out_specs=pl.BlockSpec((_SB // 4, 4 * _D), lambda i: (i, 0)),
        out_shape=jax.ShapeDtypeStruct((_GB * _SB // 4, 4 * _D), jnp.float32),
    )(zt)


_CH = 64
_NCHUNK = _TW // _CH
_ROWS = _CH * _K


def _sc_body(ss_hbm, z4_hbm, outq_hbm, outr_hbm, block_v, srow_v, g_v,
             q_v, o_v, sem):
    wid = lax.axis_index("s") * _NC + lax.axis_index("c")
    t0 = wid * _TW
    pltpu.sync_copy(ss_hbm.at[pl.ds(t0 * _K, _TW * _K)], block_v)

    lane = jnp.arange(16, dtype=jnp.int32)

    def sid_body(i, carry):
        ids = block_v[pl.ds(i * _L, _L)]
        srow_v[pl.ds(i * _L, _L)] = ((ids >> 13) << 11) + (ids & 2047)
        return carry

    lax.fori_loop(0, (_TW * _K) // _L, sid_body, 0)

    row_pat = 1 + (lane & 7)
    d_pat = [2 * v + (lane >> 3) for v in range(16)]
    j_pat = [v * _L + lane for v in range(16)]

    for c in range(_NCHUNK // 2):
        for h in range(2):
            base = (c * 2 + h) * _ROWS
            for k in range(4):
                idx = srow_v.at[pl.ds(base + k * 128, 128)]
                pltpu.async_copy(
                    z4_hbm.at[idx], g_v.at[pl.ds(k * 128, 128)], sem
                )
            idx = srow_v.at[pl.ds(base + 512, 64)]
            pltpu.async_copy(z4_hbm.at[idx], g_v.at[pl.ds(512, 64)], sem)
            pltpu.make_async_copy(
                z4_hbm.at[pl.ds(0, _ROWS)], g_v, sem
            ).wait()

            def trial_body(tl, carry):
                g_base = tl * _K
                id_base = base + g_base
                rid = plsc.load_gather(block_v, [id_base + row_pat])
                qid = plsc.load_gather(block_v, [id_base + (lane & 0)])
                rcol = ((rid >> 11) & 3) * _D
                qcol = ((qid >> 11) & 3) * _D
                col = (lane & 0) + tl + h * _CH
                for v in range(_D // _L):
                    vals = plsc.load_gather(
                        g_v, [g_base + (lane & 0), qcol + v * _L + lane]
                    )
                    plsc.store_scatter(q_v, [v * _L + lane, col], vals)
                for v in range(16):
                    vals = plsc.load_gather(
                        g_v, [g_base + row_pat, rcol + d_pat[v]]
                    )
                    plsc.store_scatter(o_v, [j_pat[v], col], vals)
                return carry

            lax.fori_loop(0, _CH, trial_body, 0)

        tc0 = t0 + c * 2 * _CH
        pltpu.sync_copy(q_v, outq_hbm.at[:, pl.ds(tc0, 2 * _CH)])
        pltpu.sync_copy(o_v, outr_hbm.at[:, pl.ds(tc0, 2 * _CH)])


@jax.jit
def _run(ss_flat, zt):
    z4 = _retile(zt)
    kfn = pl.kernel(
        _sc_body,
        out_type=(
            jax.ShapeDtypeStruct((_D, _T), jnp.float32),
            jax.ShapeDtypeStruct((_D * _R, _T), jnp.float32),
        ),
        mesh=plsc.VectorSubcoreMesh(
            core_axis_name="c", subcore_axis_name="s",
            num_cores=_NC, num_subcores=_NS,
        ),
        scratch_types=[
            pltpu.VMEM((_TW * _K,), jnp.int32),
            pltpu.VMEM((_TW * _K,), jnp.int32),
            pltpu.VMEM((_ROWS, 4 * _D), jnp.float32),
            pltpu.VMEM((_D, 2 * _CH), jnp.float32),
            pltpu.VMEM((_D * _R, 2 * _CH), jnp.float32),
            pltpu.SemaphoreType.DMA,
        ],
        compiler_params=pltpu.CompilerParams(needs_layout_passes=False),
    )
    return kfn(ss_flat, z4)


def kernel(stimulus_set, max_n_reference, z):
    del max_n_reference
    q2, r2 = _run(stimulus_set.reshape(-1), jnp.transpose(z))
    zq = jnp.transpose(q2).reshape(_T, _D, 1)
    zr = jnp.transpose(r2.reshape(_D, _R, _T), (2, 0, 1))
    return zq, zr


# speedup vs baseline: 6.0111x; 1.0101x over previous
import functools

import jax
import jax.numpy as jnp
from jax import lax
from jax.experimental import pallas as pl
from jax.experimental.pallas import tpu as pltpu
from jax.experimental.pallas import tpu_sc as plsc

_NC, _NS, _L = 2, 16, 16
_NW = _NC * _NS

_T, _K, _D = 16384, 9, 32
_R = _K - 1
_V = 1000000
_TW = _T // _NW
_TI = _TW // _L

_SB = 16384
_GB = (_V + _SB - 1) // _SB


def _retile_body(x_ref, o_ref):
    y = jax.lax.dot_general(
        x_ref[...], jnp.eye(_D, dtype=jnp.float32),
        (((0,), (0,)), ((), ())),
        preferred_element_type=jnp.float32,
    )
    o_ref[...] = jnp.concatenate(
        [y[k * (_SB // 4):(k + 1) * (_SB // 4)] for k in range(4)], axis=1
    )


def _retile(zt):
    return pl.pallas_call(
        _retile_body,
        grid=(_GB,),
        in_specs=[pl.BlockSpec((_D, _SB), lambda i: (0, i))],
        out_specs=pl.BlockSpec((_SB // 4, 4 * _D), lambda i: (i, 0)),
        out_shape=jax.ShapeDtypeStruct((_GB * _SB // 4, 4 * _D), jnp.float32),
    )(zt)


_CH = 64
_NCHUNK = _TW // _CH
_ROWS = _CH * _K


def _sc_body(ss_hbm, z4_hbm, outq_hbm, outr_hbm, block_v, srow_v, g_v,
             q_v, o_v, sem):
    wid = lax.axis_index("s") * _NC + lax.axis_index("c")
    t0 = wid * _TW
    pltpu.sync_copy(ss_hbm.at[pl.ds(t0 * _K, _TW * _K)], block_v)

    lane = jnp.arange(16, dtype=jnp.int32)

    def sid_body(i, carry):
        ids = block_v[pl.ds(i * _L, _L)]
        srow_v[pl.ds(i * _L, _L)] = ((ids >> 14) << 12) + (ids & 4095)
        return carry

    lax.fori_loop(0, (_TW * _K) // _L, sid_body, 0)

    row_pat = 1 + (lane & 7)
    d_pat = [2 * v + (lane >> 3) for v in range(16)]
    j_pat = [v * _L + lane for v in range(16)]

    for c in range(_NCHUNK // 2):
        for h in range(2):
            base = (c * 2 + h) * _ROWS
            for k in range(4):
                idx = srow_v.at[pl.ds(base + k * 128, 128)]
                pltpu.async_copy(
                    z4_hbm.at[idx], g_v.at[pl.ds(k * 128, 128)], sem
                )
            idx = srow_v.at[pl.ds(base + 512, 64)]
            pltpu.async_copy(z4_hbm.at[idx], g_v.at[pl.ds(512, 64)], sem)
            pltpu.make_async_copy(
                z4_hbm.at[pl.ds(0, _ROWS)], g_v, sem
            ).wait()

            def trial_body(tl, carry):
                g_base = tl * _K
                id_base = base + g_base
                rid = plsc.load_gather(block_v, [id_base + row_pat])
                qid = plsc.load_gather(block_v, [id_base + (lane & 0)])
                rcol = ((rid >> 12) & 3) * _D
                qcol = ((qid >> 12) & 3) * _D
                col = (lane & 0) + tl + h * _CH
                for v in range(_D // _L):
                    vals = plsc.load_gather(
                        g_v, [g_base + (lane & 0), qcol + v * _L + lane]
                    )
                    plsc.store_scatter(q_v, [v * _L + lane, col], vals)
                for v in range(16):
                    vals = plsc.load_gather(
                        g_v, [g_base + row_pat, rcol + d_pat[v]]
                    )
                    plsc.store_scatter(o_v, [j_pat[v], col], vals)
                return carry

            lax.fori_loop(0, _CH, trial_body, 0)

        tc0 = t0 + c * 2 * _CH
        pltpu.sync_copy(q_v, outq_hbm.at[:, pl.ds(tc0, 2 * _CH)])
        pltpu.sync_copy(o_v, outr_hbm.at[:, pl.ds(tc0, 2 * _CH)])


@jax.jit
def _run(ss_flat, zt):
    z4 = _retile(zt)
    kfn = pl.kernel(
        _sc_body,
        out_type=(
            jax.ShapeDtypeStruct((_D, _T), jnp.float32),
            jax.ShapeDtypeStruct((_D * _R, _T), jnp.float32),
        ),
        mesh=plsc.VectorSubcoreMesh(
            core_axis_name="c", subcore_axis_name="s",
            num_cores=_NC, num_subcores=_NS,
        ),
        scratch_types=[
            pltpu.VMEM((_TW * _K,), jnp.int32),
            pltpu.VMEM((_TW * _K,), jnp.int32),
            pltpu.VMEM((_ROWS, 4 * _D), jnp.float32),
            pltpu.VMEM((_D, 2 * _CH), jnp.float32),
            pltpu.VMEM((_D * _R, 2 * _CH), jnp.float32),
            pltpu.SemaphoreType.DMA,
        ],
        compiler_params=pltpu.CompilerParams(needs_layout_passes=False),
    )
    return kfn(ss_flat, z4)


def kernel(stimulus_set, max_n_reference, z):
    del max_n_reference
    q2, r2 = _run(stimulus_set.reshape(-1), jnp.transpose(z))
    zq = jnp.transpose(q2).reshape(_T, _D, 1)
    zr = jnp.transpose(r2.reshape(_D, _R, _T), (2, 0, 1))
    return zq, zr


# speedup vs baseline: 6.0271x; 1.0027x over previous
import functools

import jax
import jax.numpy as jnp
from jax import lax
from jax.experimental import pallas as pl
from jax.experimental.pallas import tpu as pltpu
from jax.experimental.pallas import tpu_sc as plsc

_NC, _NS, _L = 2, 16, 16
_NW = _NC * _NS

_T, _K, _D = 16384, 9, 32
_R = _K - 1
_V = 1000000
_TW = _T // _NW
_TI = _TW // _L

_SB = 16384
_GB = (_V + _SB - 1) // _SB


def _retile_body(x_ref, o_ref):
    y = x_ref[...].T
    o_ref[...] = jnp.concatenate(
        [y[k * (_SB // 4):(k + 1) * (_SB // 4)] for k in range(4)], axis=1
    )


def _retile(zt):
    return pl.pallas_call(
        _retile_body,
        grid=(_GB,),
        in_specs=[pl.BlockSpec((_D, _SB), lambda i: (0, i))],
        out_specs=pl.BlockSpec((_SB // 4, 4 * _D), lambda i: (i, 0)),
        out_shape=jax.ShapeDtypeStruct((_GB * _SB // 4, 4 * _D), jnp.float32),
    )(zt)


_CH = 64
_NCHUNK = _TW // _CH
_ROWS = _CH * _K


def _sc_body(ss_hbm, z4_hbm, outq_hbm, outr_hbm, block_v, srow_v, g_v,
             q_v, o_v, sem):
    wid = lax.axis_index("s") * _NC + lax.axis_index("c")
    t0 = wid * _TW
    pltpu.sync_copy(ss_hbm.at[pl.ds(t0 * _K, _TW * _K)], block_v)

    lane = jnp.arange(16, dtype=jnp.int32)

    def sid_body(i, carry):
        ids = block_v[pl.ds(i * _L, _L)]
        srow_v[pl.ds(i * _L, _L)] = ((ids >> 14) << 12) + (ids & 4095)
        return carry

    lax.fori_loop(0, (_TW * _K) // _L, sid_body, 0)

    row_pat = 1 + (lane & 7)
    d_pat = [2 * v + (lane >> 3) for v in range(16)]
    j_pat = [v * _L + lane for v in range(16)]

    for c in range(_NCHUNK // 2):
        for h in range(2):
            base = (c * 2 + h) * _ROWS
            for k in range(4):
                idx = srow_v.at[pl.ds(base + k * 128, 128)]
                pltpu.async_copy(
                    z4_hbm.at[idx], g_v.at[pl.ds(k * 128, 128)], sem
                )
            idx = srow_v.at[pl.ds(base + 512, 64)]
            pltpu.async_copy(z4_hbm.at[idx], g_v.at[pl.ds(512, 64)], sem)
            pltpu.make_async_copy(
                z4_hbm.at[pl.ds(0, _ROWS)], g_v, sem
            ).wait()

            def trial_body(tl, carry):
                g_base = tl * _K
                id_base = base + g_base
                rid = plsc.load_gather(block_v, [id_base + row_pat])
                qid = plsc.load_gather(block_v, [id_base + (lane & 0)])
                rcol = ((rid >> 12) & 3) * _D
                qcol = ((qid >> 12) & 3) * _D
                col = (lane & 0) + tl + h * _CH
                for v in range(_D // _L):
                    vals = plsc.load_gather(
                        g_v, [g_base + (lane & 0), qcol + v * _L + lane]
                    )
                    plsc.store_scatter(q_v, [v * _L + lane, col], vals)
                for v in range(16):
                    vals = plsc.load_gather(
                        g_v, [g_base + row_pat, rcol + d_pat[v]]
                    )
                    plsc.store_scatter(o_v, [j_pat[v], col], vals)
                return carry

            lax.fori_loop(0, _CH, trial_body, 0)

        tc0 = t0 + c * 2 * _CH
        pltpu.sync_copy(q_v, outq_hbm.at[:, pl.ds(tc0, 2 * _CH)])
        pltpu.sync_copy(o_v, outr_hbm.at[:, pl.ds(tc0, 2 * _CH)])


@jax.jit
def _run(ss_flat, zt):
    z4 = _retile(zt)
    kfn = pl.kernel(
        _sc_body,
        out_type=(
            jax.ShapeDtypeStruct((_D, _T), jnp.float32),
            jax.ShapeDtypeStruct((_D * _R, _T), jnp.float32),
        ),
        mesh=plsc.VectorSubcoreMesh(
            core_axis_name="c", subcore_axis_name="s",
            num_cores=_NC, num_subcores=_NS,
        ),
        scratch_types=[
            pltpu.VMEM((_TW * _K,), jnp.int32),
            pltpu.VMEM((_TW * _K,), jnp.int32),
            pltpu.VMEM((_ROWS, 4 * _D), jnp.float32),
            pltpu.VMEM((_D, 2 * _CH), jnp.float32),
            pltpu.VMEM((_D * _R, 2 * _CH), jnp.float32),
            pltpu.SemaphoreType.DMA,
        ],
        compiler_params=pltpu.CompilerParams(needs_layout_passes=False),
    )
    return kfn(ss_flat, z4)


def kernel(stimulus_set, max_n_reference, z):
    del max_n_reference
    q2, r2 = _run(stimulus_set.reshape(-1), jnp.transpose(z))
    zq = jnp.transpose(q2).reshape(_T, _D, 1)
    zr = jnp.transpose(r2.reshape(_D, _R, _T), (2, 0, 1))
    return zq, zr


# speedup vs baseline: 6.4039x; 1.0625x over previous
import functools

import jax
import jax.numpy as jnp
from jax import lax
from jax.experimental import pallas as pl
from jax.experimental.pallas import tpu as pltpu
from jax.experimental.pallas import tpu_sc as plsc

_NC, _NS, _L = 2, 16, 16
_NW = _NC * _NS

_T, _K, _D = 16384, 9, 32
_R = _K - 1
_V = 1000000
_TW = _T // _NW
_TI = _TW // _L

_SB = 16384
_GB = (_V + _SB - 1) // _SB


def _retile_body(x_ref, o_ref):
    y = x_ref[...].T
    o_ref[...] = jnp.concatenate(
        [y[k * (_SB // 4):(k + 1) * (_SB // 4)] for k in range(4)], axis=1
    )


def _retile(zt):
    return pl.pallas_call(
        _retile_body,
        grid=(_GB,),
        in_specs=[pl.BlockSpec((_D, _SB), lambda i: (0, i))],
        out_specs=pl.BlockSpec((_SB // 4, 4 * _D), lambda i: (i, 0)),
        out_shape=jax.ShapeDtypeStruct((_GB * _SB // 4, 4 * _D), jnp.float32),
    )(zt)


_CH = 32
_NCHUNK = _TW // _CH
_ROWS = _CH * _K


def _sc_body(ss_hbm, z4_hbm, outq_hbm, outr_hbm, block_v, srow_v, g0_v,
             g1_v, q_v, o_v, sem0, sem1):
    wid = lax.axis_index("s") * _NC + lax.axis_index("c")
    t0 = wid * _TW
    pltpu.sync_copy(ss_hbm.at[pl.ds(t0 * _K, _TW * _K)], block_v)

    lane = jnp.arange(16, dtype=jnp.int32)

    def sid_body(i, carry):
        ids = block_v[pl.ds(i * _L, _L)]
        srow_v[pl.ds(i * _L, _L)] = ((ids >> 14) << 12) + (ids & 4095)
        return carry

    lax.fori_loop(0, (_TW * _K) // _L, sid_body, 0)

    row_pat = 1 + (lane & 7)
    d_pat = [2 * v + (lane >> 3) for v in range(16)]
    j_pat = [v * _L + lane for v in range(16)]

    gbufs = (g0_v, g1_v)
    sems = (sem0, sem1)

    def fire(u):
        base = u * _ROWS
        g_v = gbufs[u % 2]
        sem = sems[u % 2]
        for k in range(2):
            idx = srow_v.at[pl.ds(base + k * 128, 128)]
            pltpu.async_copy(
                z4_hbm.at[idx], g_v.at[pl.ds(k * 128, 128)], sem
            )
        idx = srow_v.at[pl.ds(base + 256, 32)]
        pltpu.async_copy(z4_hbm.at[idx], g_v.at[pl.ds(256, 32)], sem)

    fire(0)
    for u in range(_NCHUNK):
        if u + 1 < _NCHUNK:
            fire(u + 1)
        g_v = gbufs[u % 2]
        pltpu.make_async_copy(
            z4_hbm.at[pl.ds(0, _ROWS)], g_v, sems[u % 2]
        ).wait()
        base = u * _ROWS
        cbase = (u % 4) * _CH

        def trial_body(tl, carry, g_v=g_v, base=base, cbase=cbase):
            g_base = tl * _K
            id_base = base + g_base
            rid = plsc.load_gather(block_v, [id_base + row_pat])
            qid = plsc.load_gather(block_v, [id_base + (lane & 0)])
            rcol = ((rid >> 12) & 3) * _D
            qcol = ((qid >> 12) & 3) * _D
            col = (lane & 0) + tl + cbase
            for v in range(_D // _L):
                vals = plsc.load_gather(
                    g_v, [g_base + (lane & 0), qcol + v * _L + lane]
                )
                plsc.store_scatter(q_v, [v * _L + lane, col], vals)
            for v in range(16):
                vals = plsc.load_gather(
                    g_v, [g_base + row_pat, rcol + d_pat[v]]
                )
                plsc.store_scatter(o_v, [j_pat[v], col], vals)
            return carry

        lax.fori_loop(0, _CH, trial_body, 0)

        if u % 4 == 3:
            tc0 = t0 + (u - 3) * _CH
            pltpu.sync_copy(q_v, outq_hbm.at[:, pl.ds(tc0, 4 * _CH)])
            pltpu.sync_copy(o_v, outr_hbm.at[:, pl.ds(tc0, 4 * _CH)])


@jax.jit
def _run(ss_flat, zt):
    z4 = _retile(zt)
    kfn = pl.kernel(
        _sc_body,
        out_type=(
            jax.ShapeDtypeStruct((_D, _T), jnp.float32),
            jax.ShapeDtypeStruct((_D * _R, _T), jnp.float32),
        ),
        mesh=plsc.VectorSubcoreMesh(
            core_axis_name="c", subcore_axis_name="s",
            num_cores=_NC, num_subcores=_NS,
        ),
        scratch_types=[
            pltpu.VMEM((_TW * _K,), jnp.int32),
            pltpu.VMEM((_TW * _K,), jnp.int32),
            pltpu.VMEM((_ROWS, 4 * _D), jnp.float32),
            pltpu.VMEM((_ROWS, 4 * _D), jnp.float32),
            pltpu.VMEM((_D, 4 * _CH), jnp.float32),
            pltpu.VMEM((_D * _R, 4 * _CH), jnp.float32),
            pltpu.SemaphoreType.DMA,
            pltpu.SemaphoreType.DMA,
        ],
        compiler_params=pltpu.CompilerParams(needs_layout_passes=False),
    )
    return kfn(ss_flat, z4)


def kernel(stimulus_set, max_n_reference, z):
    del max_n_reference
    q2, r2 = _run(stimulus_set.reshape(-1), jnp.transpose(z))
    zq = jnp.transpose(q2).reshape(_T, _D, 1)
    zr = jnp.transpose(r2.reshape(_D, _R, _T), (2, 0, 1))
    return zq, zr


# speedup vs baseline: 6.4042x; 1.0000x over previous
import jax
import jax.numpy as jnp
from jax import lax
from jax.experimental import pallas as pl
from jax.experimental.pallas import tpu as pltpu
from jax.experimental.pallas import tpu_sc as plsc

_NC, _NS, _L = 2, 16, 16
_NW = _NC * _NS

_T, _K, _D = 16384, 9, 32
_R = _K - 1
_V = 1000000
_TW = _T // _NW

_SB = 16384
_GB = (_V + _SB - 1) // _SB


def _retile_body(x_ref, o_ref):
    y = x_ref[...].T
    o_ref[...] = jnp.concatenate(
        [y[k * (_SB // 4):(k + 1) * (_SB // 4)] for k in range(4)], axis=1
    )


def _retile(zt):
    return pl.pallas_call(
        _retile_body,
        grid=(_GB,),
        in_specs=[pl.BlockSpec((_D, _SB), lambda i: (0, i))],
        out_specs=pl.BlockSpec((_SB // 4, 4 * _D), lambda i: (i, 0)),
        out_shape=jax.ShapeDtypeStruct((_GB * _SB // 4, 4 * _D), jnp.float32),
    )(zt)


_CH = 32
_NCHUNK = _TW // _CH
_ROWS = _CH * _K


def _sc_body(ss_hbm, z4_hbm, outq_hbm, outr_hbm, block_v, srow_v, g0_v,
             g1_v, q_v, o_v, sem0, sem1):
    wid = lax.axis_index("s") * _NC + lax.axis_index("c")
    t0 = wid * _TW
    pltpu.sync_copy(ss_hbm.at[pl.ds(t0 * _K, _TW * _K)], block_v)

    lane = jnp.arange(16, dtype=jnp.int32)

    def sid_body(i, carry):
        ids = block_v[pl.ds(i * _L, _L)]
        srow_v[pl.ds(i * _L, _L)] = ((ids >> 14) << 12) + (ids & 4095)
        return carry

    lax.fori_loop(0, (_TW * _K) // _L, sid_body, 0)

    row_pat = 1 + (lane & 7)
    d_pat = [2 * v + (lane >> 3) for v in range(16)]
    j_pat = [v * _L + lane for v in range(16)]

    gbufs = (g0_v, g1_v)
    sems = (sem0, sem1)

    def fire(u):
        base = u * _ROWS
        g_v = gbufs[u % 2]
        sem = sems[u % 2]
        for k in range(2):
            idx = srow_v.at[pl.ds(base + k * 128, 128)]
            pltpu.async_copy(
                z4_hbm.at[idx], g_v.at[pl.ds(k * 128, 128)], sem
            )
        idx = srow_v.at[pl.ds(base + 256, 32)]
        pltpu.async_copy(z4_hbm.at[idx], g_v.at[pl.ds(256, 32)], sem)

    fire(0)
    for u in range(_NCHUNK):
        if u + 1 < _NCHUNK:
            fire(u + 1)
        g_v = gbufs[u % 2]
        pltpu.make_async_copy(
            z4_hbm.at[pl.ds(0, _ROWS)], g_v, sems[u % 2]
        ).wait()
        base = u * _ROWS
        cbase = (u % 4) * _CH

        def trial_body(tl, carry, g_v=g_v, base=base, cbase=cbase):
            g_base = tl * _K
            id_base = base + g_base
            rid = plsc.load_gather(block_v, [id_base + row_pat])
            qid = plsc.load_gather(block_v, [id_base + (lane & 0)])
            rcol = ((rid >> 12) & 3) * _D
            qcol = ((qid >> 12) & 3) * _D
            col = (lane & 0) + tl + cbase
            for v in range(_D // _L):
                vals = plsc.load_gather(
                    g_v, [g_base + (lane & 0), qcol + v * _L + lane]
                )
                plsc.store_scatter(q_v, [v * _L + lane, col], vals)
            for v in range(16):
                vals = plsc.load_gather(
                    g_v, [g_base + row_pat, rcol + d_pat[v]]
                )
                plsc.store_scatter(o_v, [j_pat[v], col], vals)
            return carry

        lax.fori_loop(0, _CH, trial_body, 0)

        if u % 4 == 3:
            tc0 = t0 + (u - 3) * _CH
            pltpu.sync_copy(q_v, outq_hbm.at[:, pl.ds(tc0, 4 * _CH)])
            pltpu.sync_copy(o_v, outr_hbm.at[:, pl.ds(tc0, 4 * _CH)])


@jax.jit
def _run(ss_flat, zt):
    z4 = _retile(zt)
    kfn = pl.kernel(
        _sc_body,
        out_type=(
            jax.ShapeDtypeStruct((_D, _T), jnp.float32),
            jax.ShapeDtypeStruct((_D * _R, _T), jnp.float32),
        ),
        mesh=plsc.VectorSubcoreMesh(
            core_axis_name="c", subcore_axis_name="s",
            num_cores=_NC, num_subcores=_NS,
        ),
        scratch_types=[
            pltpu.VMEM((_TW * _K,), jnp.int32),
            pltpu.VMEM((_TW * _K,), jnp.int32),
            pltpu.VMEM((_ROWS, 4 * _D), jnp.float32),
            pltpu.VMEM((_ROWS, 4 * _D), jnp.float32),
            pltpu.VMEM((_D, 4 * _CH), jnp.float32),
            pltpu.VMEM((_D * _R, 4 * _CH), jnp.float32),
            pltpu.SemaphoreType.DMA,
            pltpu.SemaphoreType.DMA,
        ],
        compiler_params=pltpu.CompilerParams(needs_layout_passes=False),
    )
    return kfn(ss_flat, z4)


def kernel(stimulus_set, max_n_reference, z):
    del max_n_reference
    q2, r2 = _run(stimulus_set.reshape(-1), jnp.transpose(z))
    zq = jnp.transpose(q2).reshape(_T, _D, 1)
    zr = jnp.transpose(r2.reshape(_D, _R, _T), (2, 0, 1))
    return zq, zr
